# Initial kernel scaffold; baseline (speedup 1.0000x reference)
import jax
import jax.numpy as jnp
from jax.experimental import pallas as pl


def kernel(points_coords, points_labels, boxes, masks, pe_gauss, pt_emb, not_a_point_w, conv1_w, conv1_b, ln1_w, ln1_b, conv2_w, conv2_b, ln2_w, ln2_b, conv3_w, conv3_b):
    raise NotImplementedError("write your pallas kernel here")


# speedup vs baseline: 1.2049x; 1.2049x over previous
import functools

import numpy as np
import jax
import jax.numpy as jnp
from jax import lax
from jax.experimental import pallas as pl
from jax.experimental.pallas import tpu as pltpu
from jax.experimental.pallas import tpu_sc as plsc

B = 64
N_PTS = 32
N_TOK = N_PTS + 2
D = 256
NPIX = 64 * 64
NW = 32
TPW = 72
NTOK_PAD = NW * TPW
_SQRT2 = np.float32(np.sqrt(2.0))
_TWO_PI = np.float32(2.0 * np.pi)


def _gelu(x):
    return x * 0.5 * (1.0 + lax.erf(x / _SQRT2))


def _dense_body(x_ref, w1_ref, b1_ref, m1_ref, g1_ref, bg1_ref,
                w2_ref, b2_ref, g2_ref, bg2_ref, w3_ref, b3_ref, out_ref):
    x = x_ref[0]
    h = jnp.dot(w1_ref[...], x, preferred_element_type=jnp.float32) + b1_ref[...]
    u = jnp.dot(m1_ref[...], h, preferred_element_type=jnp.float32)
    v = jnp.dot(m1_ref[...], h * h, preferred_element_type=jnp.float32) - u * u
    h = _gelu((h - u) * lax.rsqrt(v + 1e-6) * g1_ref[...] + bg1_ref[...])
    h = jnp.dot(w2_ref[...], h, preferred_element_type=jnp.float32) + b2_ref[...]
    u = jnp.mean(h, axis=0, keepdims=True)
    v = jnp.mean(h * h, axis=0, keepdims=True) - u * u
    h = _gelu((h - u) * lax.rsqrt(v + 1e-6) * g2_ref[...] + bg2_ref[...])
    out_ref[0] = jnp.dot(w3_ref[...], h, preferred_element_type=jnp.float32) + b3_ref[...]


def _dense_path(masks, conv1_w, conv1_b, ln1_w, ln1_b,
                conv2_w, conv2_b, ln2_w, ln2_b, conv3_w, conv3_b):
    x16 = masks.reshape(B, 64, 4, 64, 4).transpose(0, 2, 4, 1, 3).reshape(B, 16, NPIX)

    w1 = conv1_w.reshape(4, 2, 2)
    w1full = jnp.zeros((4, 2, 2, 4, 4), jnp.float32)
    for di in range(2):
        for dj in range(2):
            w1full = w1full.at[:, di, dj, 2 * di:2 * di + 2, 2 * dj:2 * dj + 2].set(w1)
    w1big = w1full.reshape(16, 16)
    b1big = jnp.repeat(conv1_b, 4)[:, None]
    m1 = jnp.asarray(np.kron(np.ones((4, 4), np.float32) / 4.0, np.eye(4, dtype=np.float32)))
    g1 = jnp.repeat(ln1_w, 4)[:, None]
    bg1 = jnp.repeat(ln1_b, 4)[:, None]
    w2big = conv2_w.reshape(16, 16)
    b2 = conv2_b[:, None]
    g2 = ln2_w[:, None]
    bg2 = ln2_b[:, None]
    w3 = conv3_w.reshape(256, 16)
    b3 = conv3_b[:, None]

    full = lambda i: (0, 0)
    out = pl.pallas_call(
        _dense_body,
        grid=(B,),
        in_specs=[
            pl.BlockSpec((1, 16, NPIX), lambda i: (i, 0, 0)),
            pl.BlockSpec((16, 16), full), pl.BlockSpec((16, 1), full),
            pl.BlockSpec((16, 16), full), pl.BlockSpec((16, 1), full),
            pl.BlockSpec((16, 1), full),
            pl.BlockSpec((16, 16), full), pl.BlockSpec((16, 1), full),
            pl.BlockSpec((16, 1), full), pl.BlockSpec((16, 1), full),
            pl.BlockSpec((256, 16), full), pl.BlockSpec((256, 1), full),
        ],
        out_specs=pl.BlockSpec((1, 256, NPIX), lambda i: (i, 0, 0)),
        out_shape=jax.ShapeDtypeStruct((B, 256, NPIX), jnp.float32),
    )(x16, w1big, b1big, m1, g1, bg1, w2big, b2, g2, bg2, w3, b3)
    return out.reshape(B, 256, 64, 64)


def _pe_body(c_ref, g_ref, lab_ref, out_ref):
    c = c_ref[...] * 2.0 - 1.0
    t = jnp.dot(c, g_ref[...], preferred_element_type=jnp.float32) * _TWO_PI
    pe = jnp.concatenate([jnp.sin(t), jnp.cos(t)], axis=-1)
    out_ref[...] = jnp.where(lab_ref[...] == -1, 0.0, pe)


def _sc_embed_body(pe_hbm, idx_hbm, tab_hbm, out_hbm, idx_v, rows_v, pe_v, sem):
    w = lax.axis_index("s") * 2 + lax.axis_index("c")
    pltpu.sync_copy(idx_hbm.at[w], idx_v)
    pltpu.async_copy(tab_hbm.at[idx_v], rows_v, sem).wait()
    pltpu.sync_copy(pe_hbm.at[w], pe_v)

    def _row(i, carry):
        for j in range(D // 16):
            s = pl.ds(j * 16, 16)
            pe_v[i, s] = pe_v[i, s] + rows_v[i, s]
        return carry

    lax.fori_loop(0, TPW, _row, 0)
    pltpu.sync_copy(pe_v, out_hbm.at[w])


def _sparse_path(points_coords, points_labels, boxes, pe_gauss, pt_emb, not_a_point_w):
    pts = (points_coords + 0.5) * (1.0 / 1024.0)
    crn = (boxes.reshape(B, 2, 2) + 0.5) * (1.0 / 1024.0)
    coords = jnp.concatenate([pts, crn], axis=1).reshape(B * N_TOK, 2)
    coords = jnp.pad(coords, ((0, NTOK_PAD - B * N_TOK), (0, 0)))
    labels = jnp.concatenate(
        [points_labels, jnp.full((B, 2), 2, jnp.int32)], axis=1).reshape(B * N_TOK)
    labels = jnp.pad(labels, (0, NTOK_PAD - B * N_TOK), constant_values=2)

    pe = pl.pallas_call(
        _pe_body,
        in_specs=[pl.BlockSpec((NTOK_PAD, 2), lambda: (0, 0)),
                  pl.BlockSpec((2, 128), lambda: (0, 0)),
                  pl.BlockSpec((NTOK_PAD, 1), lambda: (0, 0))],
        out_specs=pl.BlockSpec((NTOK_PAD, D), lambda: (0, 0)),
        out_shape=jax.ShapeDtypeStruct((NTOK_PAD, D), jnp.float32),
    )(coords, pe_gauss, labels[:, None])

    table = jnp.concatenate(
        [not_a_point_w, pt_emb, jnp.zeros((3, D), jnp.float32)], axis=0)
    idx_pts = points_labels + 1
    idx_box = jnp.broadcast_to(jnp.array([3, 4], jnp.int32), (B, 2))
    idx = jnp.concatenate([idx_pts, idx_box], axis=1).reshape(B * N_TOK)
    idx = jnp.pad(idx, (0, NTOK_PAD - B * N_TOK)).reshape(NW, TPW)

    mesh = plsc.VectorSubcoreMesh(core_axis_name="c", subcore_axis_name="s")
    sc = functools.partial(
        pl.kernel, mesh=mesh,
        out_type=jax.ShapeDtypeStruct((NW, TPW, D), jnp.float32),
        scratch_types=[
            pltpu.VMEM((TPW,), jnp.int32),
            pltpu.VMEM((TPW, D), jnp.float32),
            pltpu.VMEM((TPW, D), jnp.float32),
            pltpu.SemaphoreType.DMA,
        ],
    )(_sc_embed_body)
    out = sc(pe.reshape(NW, TPW, D), idx, table)
    return out.reshape(NTOK_PAD, D)[:B * N_TOK].reshape(B, N_TOK, D)


def kernel(points_coords, points_labels, boxes, masks, pe_gauss, pt_emb,
           not_a_point_w, conv1_w, conv1_b, ln1_w, ln1_b, conv2_w, conv2_b,
           ln2_w, ln2_b, conv3_w, conv3_b):
    sparse = _sparse_path(points_coords, points_labels, boxes,
                          pe_gauss, pt_emb, not_a_point_w)
    dense = _dense_path(masks, conv1_w, conv1_b, ln1_w, ln1_b,
                        conv2_w, conv2_b, ln2_w, ln2_b, conv3_w, conv3_b)
    return (sparse, dense)


# speedup vs baseline: 1.2847x; 1.0663x over previous
import functools

import numpy as np
import jax
import jax.numpy as jnp
from jax import lax
from jax.experimental import pallas as pl
from jax.experimental.pallas import tpu as pltpu
from jax.experimental.pallas import tpu_sc as plsc

B = 64
N_PTS = 32
N_TOK = N_PTS + 2
D = 256
NPIX = 64 * 64
NW = 32
TPW = 72
NTOK_PAD = NW * TPW
_SQRT2 = np.float32(np.sqrt(2.0))
_TWO_PI = np.float32(2.0 * np.pi)


def _gelu(x):
    return x * 0.5 * (1.0 + lax.erf(x / _SQRT2))


def _dense_body(x_ref, cb_ref, w1_ref, b1_ref, m1_ref, g1_ref, bg1_ref,
                w2_ref, b2_ref, g2_ref, bg2_ref, w3_ref, b3_ref, out_ref):
    x3 = x_ref[0]
    slabs = []
    for oi in range(4):
        p = jnp.dot(x3[:, oi, :], cb_ref[...],
                    preferred_element_type=jnp.float32)
        for oj in range(4):
            slabs.append(p[:, oj * 64:(oj + 1) * 64])
    x = jnp.stack(slabs, axis=0).reshape(16, 4096)
    h = jnp.dot(w1_ref[...], x, preferred_element_type=jnp.float32) + b1_ref[...]
    u = jnp.dot(m1_ref[...], h, preferred_element_type=jnp.float32)
    v = jnp.dot(m1_ref[...], h * h, preferred_element_type=jnp.float32) - u * u
    h = _gelu((h - u) * lax.rsqrt(v + 1e-6) * g1_ref[...] + bg1_ref[...])
    h = jnp.dot(w2_ref[...], h, preferred_element_type=jnp.float32) + b2_ref[...]
    u = jnp.mean(h, axis=0, keepdims=True)
    v = jnp.mean(h * h, axis=0, keepdims=True) - u * u
    h = _gelu((h - u) * lax.rsqrt(v + 1e-6) * g2_ref[...] + bg2_ref[...])
    out_ref[0] = jnp.dot(w3_ref[...], h, preferred_element_type=jnp.float32) + b3_ref[...]


def _dense_path(masks, conv1_w, conv1_b, ln1_w, ln1_b,
                conv2_w, conv2_b, ln2_w, ln2_b, conv3_w, conv3_b):
    x4 = masks.reshape(B, 64, 4, 256)
    cb = np.zeros((256, 256), np.float32)
    for oj in range(4):
        for j2 in range(64):
            cb[4 * j2 + oj, oj * 64 + j2] = 1.0
    cb = jnp.asarray(cb)

    w1 = conv1_w.reshape(4, 2, 2)
    w1full = jnp.zeros((4, 2, 2, 4, 4), jnp.float32)
    for di in range(2):
        for dj in range(2):
            w1full = w1full.at[:, di, dj, 2 * di:2 * di + 2, 2 * dj:2 * dj + 2].set(w1)
    w1big = w1full.reshape(16, 16)
    b1big = jnp.repeat(conv1_b, 4)[:, None]
    m1 = jnp.asarray(np.kron(np.ones((4, 4), np.float32) / 4.0, np.eye(4, dtype=np.float32)))
    g1 = jnp.repeat(ln1_w, 4)[:, None]
    bg1 = jnp.repeat(ln1_b, 4)[:, None]
    w2big = conv2_w.reshape(16, 16)
    b2 = conv2_b[:, None]
    g2 = ln2_w[:, None]
    bg2 = ln2_b[:, None]
    w3 = conv3_w.reshape(256, 16)
    b3 = conv3_b[:, None]

    full = lambda i: (0, 0)
    out = pl.pallas_call(
        _dense_body,
        grid=(B,),
        in_specs=[
            pl.BlockSpec((1, 64, 4, 256), lambda i: (i, 0, 0, 0)),
            pl.BlockSpec((256, 256), full),
            pl.BlockSpec((16, 16), full), pl.BlockSpec((16, 1), full),
            pl.BlockSpec((16, 16), full), pl.BlockSpec((16, 1), full),
            pl.BlockSpec((16, 1), full),
            pl.BlockSpec((16, 16), full), pl.BlockSpec((16, 1), full),
            pl.BlockSpec((16, 1), full), pl.BlockSpec((16, 1), full),
            pl.BlockSpec((256, 16), full), pl.BlockSpec((256, 1), full),
        ],
        out_specs=pl.BlockSpec((1, 256, NPIX), lambda i: (i, 0, 0)),
        out_shape=jax.ShapeDtypeStruct((B, 256, NPIX), jnp.float32),
    )(x4, cb, w1big, b1big, m1, g1, bg1, w2big, b2, g2, bg2, w3, b3)
    return out.reshape(B, 256, 64, 64)


def _pe_body(c_ref, g_ref, lab_ref, out_ref):
    c = c_ref[...] * 2.0 - 1.0
    t = jnp.dot(c, g_ref[...], preferred_element_type=jnp.float32) * _TWO_PI
    pe = jnp.concatenate([jnp.sin(t), jnp.cos(t)], axis=-1)
    out_ref[...] = jnp.where(lab_ref[...] == -1, 0.0, pe)


def _sc_embed_body(pe_hbm, idx_hbm, tab_hbm, out_hbm, idx_v, rows_v, pe_v, sem):
    w = lax.axis_index("s") * 2 + lax.axis_index("c")
    pltpu.sync_copy(idx_hbm.at[w], idx_v)
    pltpu.async_copy(tab_hbm.at[idx_v], rows_v, sem).wait()
    pltpu.sync_copy(pe_hbm.at[w], pe_v)

    def _row(i, carry):
        for j in range(D // 16):
            s = pl.ds(j * 16, 16)
            pe_v[i, s] = pe_v[i, s] + rows_v[i, s]
        return carry

    lax.fori_loop(0, TPW, _row, 0)
    pltpu.sync_copy(pe_v, out_hbm.at[w])


def _sparse_path(points_coords, points_labels, boxes, pe_gauss, pt_emb, not_a_point_w):
    pts = (points_coords + 0.5) * (1.0 / 1024.0)
    crn = (boxes.reshape(B, 2, 2) + 0.5) * (1.0 / 1024.0)
    coords = jnp.concatenate([pts, crn], axis=1).reshape(B * N_TOK, 2)
    coords = jnp.pad(coords, ((0, NTOK_PAD - B * N_TOK), (0, 0)))
    labels = jnp.concatenate(
        [points_labels, jnp.full((B, 2), 2, jnp.int32)], axis=1).reshape(B * N_TOK)
    labels = jnp.pad(labels, (0, NTOK_PAD - B * N_TOK), constant_values=2)

    pe = pl.pallas_call(
        _pe_body,
        in_specs=[pl.BlockSpec((NTOK_PAD, 2), lambda: (0, 0)),
                  pl.BlockSpec((2, 128), lambda: (0, 0)),
                  pl.BlockSpec((NTOK_PAD, 1), lambda: (0, 0))],
        out_specs=pl.BlockSpec((NTOK_PAD, D), lambda: (0, 0)),
        out_shape=jax.ShapeDtypeStruct((NTOK_PAD, D), jnp.float32),
    )(coords, pe_gauss, labels[:, None])

    table = jnp.concatenate(
        [not_a_point_w, pt_emb, jnp.zeros((3, D), jnp.float32)], axis=0)
    idx_pts = points_labels + 1
    idx_box = jnp.broadcast_to(jnp.array([3, 4], jnp.int32), (B, 2))
    idx = jnp.concatenate([idx_pts, idx_box], axis=1).reshape(B * N_TOK)
    idx = jnp.pad(idx, (0, NTOK_PAD - B * N_TOK)).reshape(NW, TPW)

    mesh = plsc.VectorSubcoreMesh(core_axis_name="c", subcore_axis_name="s")
    sc = functools.partial(
        pl.kernel, mesh=mesh,
        out_type=jax.ShapeDtypeStruct((NW, TPW, D), jnp.float32),
        scratch_types=[
            pltpu.VMEM((TPW,), jnp.int32),
            pltpu.VMEM((TPW, D), jnp.float32),
            pltpu.VMEM((TPW, D), jnp.float32),
            pltpu.SemaphoreType.DMA,
        ],
    )(_sc_embed_body)
    out = sc(pe.reshape(NW, TPW, D), idx, table)
    return out.reshape(NTOK_PAD, D)[:B * N_TOK].reshape(B, N_TOK, D)


def kernel(points_coords, points_labels, boxes, masks, pe_gauss, pt_emb,
           not_a_point_w, conv1_w, conv1_b, ln1_w, ln1_b, conv2_w, conv2_b,
           ln2_w, ln2_b, conv3_w, conv3_b):
    sparse = _sparse_path(points_coords, points_labels, boxes,
                          pe_gauss, pt_emb, not_a_point_w)
    dense = _dense_path(masks, conv1_w, conv1_b, ln1_w, ln1_b,
                        conv2_w, conv2_b, ln2_w, ln2_b, conv3_w, conv3_b)
    return (sparse, dense)
